# baseline (speedup 1.0000x reference)
import jax
import jax.numpy as jnp
from jax import lax
from jax.experimental import pallas as pl
from jax.experimental.pallas import tpu as pltpu
from jax.experimental.pallas import tpu_sc as plsc

B, C, H, W = 2, 256, 64, 64
N = 5000
AH = AW = 7
PB = AH * AW
SCALE = 0.0625
NC, NS = 2, 16
NWORK = NC * NS
RPT = 160
N_PAD = NWORK * RPT
RCH = 1
CPT = RPT // RCH
PTS = RCH * PB
OROW = C * PB
LANES = 16
PTSP = -(-PTS // LANES) * LANES


def _sc_body(ft_hbm, rois_hbm, out_hbm,
             roisv, idx0, idx1, idx2, idx3, wb0, wb1, wb2, wb3,
             rows0, rows1, rows2, rows3, outt, sem):
    wid = lax.axis_index("s") * NC + lax.axis_index("c")
    pltpu.sync_copy(rois_hbm.at[pl.ds(wid * RPT * 5, RPT * 5)], roisv)
    lane = lax.broadcasted_iota(jnp.int32, (LANES,), 0)

    def chunk_body(ci, carry):
        rbase_g = wid * RPT + ci * RCH

        @pl.when(rbase_g < N)
        def _():
            for g in range(-(-PTS // LANES)):
                p = lane + g * LANES
                pc = jnp.minimum(p, PTS - 1)
                rl = lax.div(pc, PB)
                rem = pc - rl * PB
                ph = lax.div(rem, AW)
                pw = rem - ph * AW
                i5 = (ci * RCH + rl) * 5
                bf = plsc.load_gather(roisv, [i5])
                x1 = plsc.load_gather(roisv, [i5 + 1]) * SCALE
                y1 = plsc.load_gather(roisv, [i5 + 2]) * SCALE
                x2 = plsc.load_gather(roisv, [i5 + 3]) * SCALE
                y2 = plsc.load_gather(roisv, [i5 + 4]) * SCALE
                bw = jnp.maximum(x2 - x1, 0.0) * (1.0 / (AW - 1))
                bh = jnp.maximum(y2 - y1, 0.0) * (1.0 / (AH - 1))
                hf = y1 + ph.astype(jnp.float32) * bh
                wf = x1 + pw.astype(jnp.float32) * bw
                valid = (hf >= 0.0) & (hf < float(H)) & (wf >= 0.0) & (wf < float(W))
                h0 = jnp.clip(hf, 0.0, float(H - 1)).astype(jnp.int32)
                w0 = jnp.clip(wf, 0.0, float(W - 1)).astype(jnp.int32)
                lh = hf - h0.astype(jnp.float32)
                lw = wf - w0.astype(jnp.float32)
                h1 = jnp.minimum(h0 + 1, H - 1)
                w1 = jnp.minimum(w0 + 1, W - 1)
                rowb = bf.astype(jnp.int32) * (H * W)
                r0 = rowb + h0 * W
                r1 = rowb + h1 * W
                vf = jnp.where(valid, 1.0, 0.0).astype(jnp.float32)
                olh = (1.0 - lh) * vf
                vlh = lh * vf
                olw = 1.0 - lw
                sl = pl.ds(g * LANES, LANES)
                idx0[sl] = r0 + w0
                idx1[sl] = r0 + w1
                idx2[sl] = r1 + w0
                idx3[sl] = r1 + w1
                wb0[sl] = olh * olw
                wb1[sl] = olh * lw
                wb2[sl] = vlh * olw
                wb3[sl] = vlh * lw

            d0 = pltpu.async_copy(ft_hbm.at[idx0], rows0, sem)
            d1 = pltpu.async_copy(ft_hbm.at[idx1], rows1, sem)
            d2 = pltpu.async_copy(ft_hbm.at[idx2], rows2, sem)
            d3 = pltpu.async_copy(ft_hbm.at[idx3], rows3, sem)
            d0.wait()
            d1.wait()
            d2.wait()
            d3.wait()

            for r in range(RCH):
                pts, msk, wv = [], [], []
                for pg in range(4):
                    pt = lane + pg * LANES
                    ptc = jnp.minimum(pt, PB - 1)
                    pr = r * PB + ptc
                    pts.append((pr, ptc))
                    msk.append(pt < PB)
                    wv.append((plsc.load_gather(wb0, [pr]),
                               plsc.load_gather(wb1, [pr]),
                               plsc.load_gather(wb2, [pr]),
                               plsc.load_gather(wb3, [pr])))

                def cbody(c, ccarry):
                    cl = jnp.zeros((LANES,), jnp.int32) + c
                    for pg in range(4):
                        pr, ptc = pts[pg]
                        w0v, w1v, w2v, w3v = wv[pg]
                        acc = (w0v * plsc.load_gather(rows0, [pr, cl])
                               + w1v * plsc.load_gather(rows1, [pr, cl])
                               + w2v * plsc.load_gather(rows2, [pr, cl])
                               + w3v * plsc.load_gather(rows3, [pr, cl]))
                        dst = (r * C + c) * PB + ptc
                        plsc.store_scatter(outt, [dst], acc, mask=msk[pg])
                    return ccarry

                lax.fori_loop(0, C, cbody, 0)

            pltpu.sync_copy(outt, out_hbm.at[pl.ds(rbase_g * OROW, RCH * OROW)])

        return carry

    lax.fori_loop(0, CPT, chunk_body, 0)


@jax.jit
def kernel(features, rois):
    ft = jnp.transpose(features, (0, 2, 3, 1)).reshape(B * H * W, C)
    rois_flat = jnp.concatenate(
        [rois.reshape(-1), jnp.zeros((N_PAD - N) * 5, jnp.float32)])
    mesh = plsc.VectorSubcoreMesh(core_axis_name="c", subcore_axis_name="s",
                                  num_cores=NC, num_subcores=NS)
    out = pl.kernel(
        _sc_body,
        out_type=jax.ShapeDtypeStruct((N * OROW,), jnp.float32),
        mesh=mesh,
        compiler_params=pltpu.CompilerParams(needs_layout_passes=False),
        scratch_types=[
            pltpu.VMEM((RPT * 5,), jnp.float32),
            pltpu.VMEM((PTSP,), jnp.int32),
            pltpu.VMEM((PTSP,), jnp.int32),
            pltpu.VMEM((PTSP,), jnp.int32),
            pltpu.VMEM((PTSP,), jnp.int32),
            pltpu.VMEM((PTSP,), jnp.float32),
            pltpu.VMEM((PTSP,), jnp.float32),
            pltpu.VMEM((PTSP,), jnp.float32),
            pltpu.VMEM((PTSP,), jnp.float32),
            pltpu.VMEM((PTSP, C), jnp.float32),
            pltpu.VMEM((PTSP, C), jnp.float32),
            pltpu.VMEM((PTSP, C), jnp.float32),
            pltpu.VMEM((PTSP, C), jnp.float32),
            pltpu.VMEM((RCH * OROW,), jnp.float32),
            pltpu.SemaphoreType.DMA,
        ],
    )(ft, rois_flat)
    return out.reshape(N, C, AH, AW)


# speedup vs baseline: 1.7661x; 1.7661x over previous
import jax
import jax.numpy as jnp
from jax import lax
from jax.experimental import pallas as pl
from jax.experimental.pallas import tpu as pltpu
from jax.experimental.pallas import tpu_sc as plsc

B, C, H, W = 2, 256, 64, 64
N = 5000
AH = AW = 7
PB = AH * AW
SCALE = 0.0625
NC, NS = 2, 16
NWORK = NC * NS
RPT = 160
N_PAD = NWORK * RPT
RCH = 1
CPT = RPT // RCH
PTS = RCH * PB
OROW = C * PB
LANES = 16
PTSP = -(-PTS // LANES) * LANES


def _sc_body(ft_hbm, rois_hbm, out_hbm,
             roisv, idx0, idx1, idx2, idx3, wb0, wb1, wb2, wb3,
             rows0, rows1, rows2, rows3, outt, sem):
    wid = lax.axis_index("s") * NC + lax.axis_index("c")
    pltpu.sync_copy(rois_hbm.at[pl.ds(wid * RPT * 5, RPT * 5)], roisv)
    lane = lax.broadcasted_iota(jnp.int32, (LANES,), 0)

    def chunk_body(ci, carry):
        rbase_g = wid * RPT + ci * RCH

        @pl.when(rbase_g < N)
        def _():
            for g in range(-(-PTS // LANES)):
                p = lane + g * LANES
                pc = jnp.minimum(p, PTS - 1)
                rl = lax.div(pc, PB)
                rem = pc - rl * PB
                ph = lax.div(rem, AW)
                pw = rem - ph * AW
                i5 = (ci * RCH + rl) * 5
                bf = plsc.load_gather(roisv, [i5])
                x1 = plsc.load_gather(roisv, [i5 + 1]) * SCALE
                y1 = plsc.load_gather(roisv, [i5 + 2]) * SCALE
                x2 = plsc.load_gather(roisv, [i5 + 3]) * SCALE
                y2 = plsc.load_gather(roisv, [i5 + 4]) * SCALE
                bw = jnp.maximum(x2 - x1, 0.0) * (1.0 / (AW - 1))
                bh = jnp.maximum(y2 - y1, 0.0) * (1.0 / (AH - 1))
                hf = y1 + ph.astype(jnp.float32) * bh
                wf = x1 + pw.astype(jnp.float32) * bw
                valid = (hf >= 0.0) & (hf < float(H)) & (wf >= 0.0) & (wf < float(W))
                h0 = jnp.clip(hf, 0.0, float(H - 1)).astype(jnp.int32)
                w0 = jnp.clip(wf, 0.0, float(W - 1)).astype(jnp.int32)
                lh = hf - h0.astype(jnp.float32)
                lw = wf - w0.astype(jnp.float32)
                h1 = jnp.minimum(h0 + 1, H - 1)
                w1 = jnp.minimum(w0 + 1, W - 1)
                rowb = bf.astype(jnp.int32) * (H * W)
                r0 = rowb + h0 * W
                r1 = rowb + h1 * W
                vf = jnp.where(valid, 1.0, 0.0).astype(jnp.float32)
                olh = (1.0 - lh) * vf
                vlh = lh * vf
                olw = 1.0 - lw
                sl = pl.ds(g * LANES, LANES)
                idx0[sl] = r0 + w0
                idx1[sl] = r0 + w1
                idx2[sl] = r1 + w0
                idx3[sl] = r1 + w1
                wb0[sl] = olh * olw
                wb1[sl] = olh * lw
                wb2[sl] = vlh * olw
                wb3[sl] = vlh * lw

            d0 = pltpu.async_copy(ft_hbm.at[idx0], rows0, sem)
            d1 = pltpu.async_copy(ft_hbm.at[idx1], rows1, sem)
            d2 = pltpu.async_copy(ft_hbm.at[idx2], rows2, sem)
            d3 = pltpu.async_copy(ft_hbm.at[idx3], rows3, sem)
            d0.wait()
            d1.wait()
            d2.wait()
            d3.wait()

            lane49 = lane * PB

            def ptbody(pt, ccarry):
                jj = jnp.zeros((LANES,), jnp.int32) + pt
                wv0 = plsc.load_gather(wb0, [jj])
                wv1 = plsc.load_gather(wb1, [jj])
                wv2 = plsc.load_gather(wb2, [jj])
                wv3 = plsc.load_gather(wb3, [jj])
                for cb in range(C // LANES):
                    cs = pl.ds(cb * LANES, LANES)
                    acc = (wv0 * rows0[pt, cs] + wv1 * rows1[pt, cs]
                           + wv2 * rows2[pt, cs] + wv3 * rows3[pt, cs])
                    dst = lane49 + (cb * LANES * PB + pt)
                    plsc.store_scatter(outt, [dst], acc)
                return ccarry

            lax.fori_loop(0, PB, ptbody, 0)

            pltpu.sync_copy(outt, out_hbm.at[pl.ds(rbase_g * OROW, RCH * OROW)])

        return carry

    lax.fori_loop(0, CPT, chunk_body, 0)


@jax.jit
def kernel(features, rois):
    ft = jnp.transpose(features, (0, 2, 3, 1)).reshape(B * H * W, C)
    rois_flat = jnp.concatenate(
        [rois.reshape(-1), jnp.zeros((N_PAD - N) * 5, jnp.float32)])
    mesh = plsc.VectorSubcoreMesh(core_axis_name="c", subcore_axis_name="s",
                                  num_cores=NC, num_subcores=NS)
    out = pl.kernel(
        _sc_body,
        out_type=jax.ShapeDtypeStruct((N * OROW,), jnp.float32),
        mesh=mesh,
        compiler_params=pltpu.CompilerParams(needs_layout_passes=False),
        scratch_types=[
            pltpu.VMEM((RPT * 5,), jnp.float32),
            pltpu.VMEM((PTSP,), jnp.int32),
            pltpu.VMEM((PTSP,), jnp.int32),
            pltpu.VMEM((PTSP,), jnp.int32),
            pltpu.VMEM((PTSP,), jnp.int32),
            pltpu.VMEM((PTSP,), jnp.float32),
            pltpu.VMEM((PTSP,), jnp.float32),
            pltpu.VMEM((PTSP,), jnp.float32),
            pltpu.VMEM((PTSP,), jnp.float32),
            pltpu.VMEM((PTSP, C), jnp.float32),
            pltpu.VMEM((PTSP, C), jnp.float32),
            pltpu.VMEM((PTSP, C), jnp.float32),
            pltpu.VMEM((PTSP, C), jnp.float32),
            pltpu.VMEM((RCH * OROW,), jnp.float32),
            pltpu.SemaphoreType.DMA,
        ],
    )(ft, rois_flat)
    return out.reshape(N, C, AH, AW)


# speedup vs baseline: 5.1710x; 2.9280x over previous
import functools

import jax
import jax.numpy as jnp
from jax import lax
from jax.experimental import pallas as pl
from jax.experimental.pallas import tpu as pltpu
from jax.experimental.pallas import tpu_sc as plsc

B, C, H, W = 2, 256, 64, 64
N = 5000
AH = AW = 7
SCALE = 0.0625
NPTS = N * AH * AW
NC, NS = 2, 16
NWORK = NC * NS
CH = 64
CPW = -(-NPTS // (NWORK * CH))
PPW = CPW * CH
NP_PAD = NWORK * PPW
LANES = 16


def _sc_body(ft_hbm, rois_hbm, out_hbm,
             rois_v, idx0, idx1, idx2, idx3, wb0, wb1, wb2, wb3,
             rows0, rows1, rows2, rows3, outb, sem):
    wid = lax.axis_index("s") * NC + lax.axis_index("c")
    pltpu.sync_copy(rois_hbm, rois_v)
    base = wid * PPW
    lane = lax.broadcasted_iota(jnp.int32, (LANES,), 0)

    def chunk_body(ci, carry):
        p0 = base + ci * CH
        for g in range(CH // LANES):
            p = lane + (p0 + g * LANES)
            n = lax.div(p, 49)
            rem = p - n * 49
            ph = lax.div(rem, 7)
            pw = rem - ph * 7
            n = jnp.minimum(n, N - 1)
            i5 = n * 5
            bf = plsc.load_gather(rois_v, [i5])
            x1 = plsc.load_gather(rois_v, [i5 + 1]) * SCALE
            y1 = plsc.load_gather(rois_v, [i5 + 2]) * SCALE
            x2 = plsc.load_gather(rois_v, [i5 + 3]) * SCALE
            y2 = plsc.load_gather(rois_v, [i5 + 4]) * SCALE
            bw = jnp.maximum(x2 - x1, 0.0) * (1.0 / (AW - 1))
            bh = jnp.maximum(y2 - y1, 0.0) * (1.0 / (AH - 1))
            hf = y1 + ph.astype(jnp.float32) * bh
            wf = x1 + pw.astype(jnp.float32) * bw
            valid = (hf >= 0.0) & (hf < float(H)) & (wf >= 0.0) & (wf < float(W))
            h0 = jnp.clip(hf, 0.0, float(H - 1)).astype(jnp.int32)
            w0 = jnp.clip(wf, 0.0, float(W - 1)).astype(jnp.int32)
            lh = hf - h0.astype(jnp.float32)
            lw = wf - w0.astype(jnp.float32)
            h1 = jnp.minimum(h0 + 1, H - 1)
            w1 = jnp.minimum(w0 + 1, W - 1)
            rowb = bf.astype(jnp.int32) * (H * W)
            r0 = rowb + h0 * W
            r1 = rowb + h1 * W
            vf = jnp.where(valid, 1.0, 0.0).astype(jnp.float32)
            olh = (1.0 - lh) * vf
            olw = 1.0 - lw
            sl = pl.ds(g * LANES, LANES)
            idx0[sl] = r0 + w0
            idx1[sl] = r0 + w1
            idx2[sl] = r1 + w0
            idx3[sl] = r1 + w1
            wb0[sl] = olh * olw
            wb1[sl] = olh * lw
            wb2[sl] = lh * vf * olw
            wb3[sl] = lh * vf * lw
        d0 = pltpu.async_copy(ft_hbm.at[idx0], rows0, sem)
        d1 = pltpu.async_copy(ft_hbm.at[idx1], rows1, sem)
        d2 = pltpu.async_copy(ft_hbm.at[idx2], rows2, sem)
        d3 = pltpu.async_copy(ft_hbm.at[idx3], rows3, sem)
        d0.wait()
        d1.wait()
        d2.wait()
        d3.wait()

        def point_body(j, jcarry):
            jj = jnp.zeros((LANES,), jnp.int32) + j
            wv0 = plsc.load_gather(wb0, [jj])
            wv1 = plsc.load_gather(wb1, [jj])
            wv2 = plsc.load_gather(wb2, [jj])
            wv3 = plsc.load_gather(wb3, [jj])
            for cb in range(C // LANES):
                cs = pl.ds(cb * LANES, LANES)
                acc = (wv0 * rows0[j, cs] + wv1 * rows1[j, cs]
                       + wv2 * rows2[j, cs] + wv3 * rows3[j, cs])
                outb[j, cs] = acc
            return jcarry

        lax.fori_loop(0, CH, point_body, 0)
        pltpu.sync_copy(outb, out_hbm.at[pl.ds(p0, CH)])
        return carry

    lax.fori_loop(0, CPW, chunk_body, 0)


PB = AH * AW
TG = 8


def _tc_transpose_body(x_ref, o_ref):
    x = x_ref[...]
    o_ref[...] = jnp.transpose(x.reshape(TG, PB, C), (0, 2, 1))


@jax.jit
def kernel(features, rois):
    ft = jnp.transpose(features, (0, 2, 3, 1)).reshape(B * H * W, C)
    rois_flat = rois.reshape(-1)
    mesh = plsc.VectorSubcoreMesh(core_axis_name="c", subcore_axis_name="s",
                                  num_cores=NC, num_subcores=NS)
    out = pl.kernel(
        _sc_body,
        out_type=jax.ShapeDtypeStruct((NP_PAD, C), jnp.float32),
        mesh=mesh,
        compiler_params=pltpu.CompilerParams(needs_layout_passes=False),
        scratch_types=[
            pltpu.VMEM((N * 5,), jnp.float32),
            pltpu.VMEM((CH,), jnp.int32),
            pltpu.VMEM((CH,), jnp.int32),
            pltpu.VMEM((CH,), jnp.int32),
            pltpu.VMEM((CH,), jnp.int32),
            pltpu.VMEM((CH,), jnp.float32),
            pltpu.VMEM((CH,), jnp.float32),
            pltpu.VMEM((CH,), jnp.float32),
            pltpu.VMEM((CH,), jnp.float32),
            pltpu.VMEM((CH, C), jnp.float32),
            pltpu.VMEM((CH, C), jnp.float32),
            pltpu.VMEM((CH, C), jnp.float32),
            pltpu.VMEM((CH, C), jnp.float32),
            pltpu.VMEM((CH, C), jnp.float32),
            pltpu.SemaphoreType.DMA,
        ],
    )(ft, rois_flat)
    outt = pl.pallas_call(
        _tc_transpose_body,
        grid=(N // TG,),
        in_specs=[pl.BlockSpec((TG * PB, C), lambda i: (i, 0))],
        out_specs=pl.BlockSpec((TG, C, PB), lambda i: (i, 0, 0)),
        out_shape=jax.ShapeDtypeStruct((N, C, PB), jnp.float32),
    )(out)
    return outt.reshape(N, C, AH, AW)
